# baseline (speedup 1.0000x reference)
import functools

import jax
import jax.numpy as jnp
from jax import lax
from jax.experimental import pallas as pl
from jax.experimental.pallas import tpu as pltpu
from jax.experimental.pallas import tpu_sc as plsc

N_NODES = 10000
N_EDGES = 320000
D_HID = 16

NUM_CORES = 2
NUM_SUBCORES = 16
NUM_WORKERS = NUM_CORES * NUM_SUBCORES
CHUNK = 2500
REAL_CHUNKS = N_EDGES // CHUNK
K = REAL_CHUNKS // NUM_WORKERS
ACC_ROWS = 10112
Z_ROWS = ACC_ROWS // NUM_SUBCORES

_MESH = plsc.VectorSubcoreMesh(core_axis_name="c", subcore_axis_name="s")


@functools.partial(
    pl.kernel,
    out_type=jax.ShapeDtypeStruct((NUM_CORES, ACC_ROWS, D_HID), jnp.float32),
    mesh=_MESH,
    scratch_types=[
        pltpu.VMEM_SHARED((ACC_ROWS, D_HID), jnp.float32),
        pltpu.VMEM((K, CHUNK), jnp.int32),
        pltpu.VMEM((K, CHUNK), jnp.int32),
        pltpu.VMEM((CHUNK, D_HID), jnp.float32),
        pltpu.VMEM((CHUNK, D_HID), jnp.float32),
        pltpu.VMEM((Z_ROWS, D_HID), jnp.float32),
        pltpu.SemaphoreType.DMA,
        pltpu.SemaphoreType.DMA,
    ],
    compiler_params=pltpu.CompilerParams(use_tc_tiling_on_sc=False),
)
def _spmm(table, edges, out, acc, src_v, dst_v, rows_a, rows_b, zero_v,
          sem_a, sem_b):
    cid = lax.axis_index("c")
    sid = lax.axis_index("s")
    wid = cid * NUM_SUBCORES + sid

    base = wid * K

    pltpu.async_copy(edges.at[0, pl.ds(base, K)], src_v, sem_a)
    pltpu.async_copy(edges.at[1, pl.ds(base, K)], dst_v, sem_b)

    def _zero(i, carry):
        zero_v[i, :] = jnp.zeros((D_HID,), jnp.float32)
        return carry

    lax.fori_loop(0, Z_ROWS, _zero, 0, unroll=8)
    pltpu.sync_copy(zero_v, acc.at[pl.ds(sid * Z_ROWS, Z_ROWS)])

    pltpu.make_async_copy(edges.at[0, pl.ds(base, K)], src_v, sem_a).wait()
    pltpu.make_async_copy(edges.at[1, pl.ds(base, K)], dst_v, sem_b).wait()
    plsc.subcore_barrier()

    pltpu.async_copy(table.at[src_v.at[0]], rows_a, sem_a)

    def _pair(i, carry):
        j0 = 2 * i
        j1 = j0 + 1
        pltpu.async_copy(table.at[src_v.at[j1]], rows_b, sem_b)
        pltpu.make_async_copy(table.at[src_v.at[j0]], rows_a, sem_a).wait()
        pltpu.sync_copy(rows_a, acc.at[dst_v.at[j0]], add=True)

        @pl.when(j0 + 2 < K)
        def _():
            pltpu.async_copy(table.at[src_v.at[j0 + 2]], rows_a, sem_a)

        pltpu.make_async_copy(table.at[src_v.at[j1]], rows_b, sem_b).wait()
        pltpu.sync_copy(rows_b, acc.at[dst_v.at[j1]], add=True)
        return carry

    lax.fori_loop(0, K // 2, _pair, 0)

    plsc.subcore_barrier()
    pltpu.sync_copy(
        acc.at[pl.ds(sid * Z_ROWS, Z_ROWS)],
        out.at[cid, pl.ds(sid * Z_ROWS, Z_ROWS)],
    )


@functools.partial(
    pl.kernel,
    out_type=[
        jax.ShapeDtypeStruct((NUM_CORES, ACC_ROWS, D_HID), jnp.float32),
        jax.ShapeDtypeStruct((NUM_CORES, ACC_ROWS, D_HID), jnp.float32),
    ],
    mesh=_MESH,
    scratch_types=[
        pltpu.VMEM_SHARED((ACC_ROWS, D_HID), jnp.float32),
        pltpu.VMEM((K, CHUNK), jnp.int32),
        pltpu.VMEM((K, CHUNK), jnp.int32),
        pltpu.VMEM((CHUNK, D_HID), jnp.float32),
        pltpu.VMEM((CHUNK, D_HID), jnp.float32),
        pltpu.VMEM((Z_ROWS, D_HID), jnp.float32),
        pltpu.VMEM((Z_ROWS, D_HID), jnp.float32),
        pltpu.VMEM((1, D_HID), jnp.float32),
        pltpu.SemaphoreType.DMA,
        pltpu.SemaphoreType.DMA,
    ],
    compiler_params=pltpu.CompilerParams(use_tc_tiling_on_sc=False),
)
def _spmm_relu(part1, edges, b1, out, htab, acc, src_v, dst_v, rows_a, rows_b,
               va, vb, b1_v, sem_a, sem_b):
    cid = lax.axis_index("c")
    sid = lax.axis_index("s")
    wid = cid * NUM_SUBCORES + sid

    base = wid * K
    pltpu.async_copy(edges.at[0, pl.ds(base, K)], src_v, sem_a)
    pltpu.async_copy(edges.at[1, pl.ds(base, K)], dst_v, sem_b)

    rows = pl.ds(sid * Z_ROWS, Z_ROWS)
    pltpu.sync_copy(b1.at[pl.ds(0, 1)], b1_v)
    pltpu.sync_copy(part1.at[0, rows], va)
    pltpu.sync_copy(part1.at[1, rows], vb)
    b1v = b1_v[0, :]

    def _mid(i, carry):
        vb[i, :] = jnp.maximum(va[i, :] + vb[i, :] + b1v, 0.0)
        va[i, :] = jnp.zeros((D_HID,), jnp.float32)
        return carry

    lax.fori_loop(0, Z_ROWS, _mid, 0, unroll=8)
    pltpu.sync_copy(vb, htab.at[cid, rows])
    pltpu.sync_copy(va, acc.at[rows])

    pltpu.make_async_copy(edges.at[0, pl.ds(base, K)], src_v, sem_a).wait()
    pltpu.make_async_copy(edges.at[1, pl.ds(base, K)], dst_v, sem_b).wait()
    plsc.subcore_barrier()

    table = htab.at[cid]
    pltpu.async_copy(table.at[src_v.at[0]], rows_a, sem_a)

    def _pair(i, carry):
        j0 = 2 * i
        j1 = j0 + 1
        pltpu.async_copy(table.at[src_v.at[j1]], rows_b, sem_b)
        pltpu.make_async_copy(table.at[src_v.at[j0]], rows_a, sem_a).wait()
        pltpu.sync_copy(rows_a, acc.at[dst_v.at[j0]], add=True)

        @pl.when(j0 + 2 < K)
        def _():
            pltpu.async_copy(table.at[src_v.at[j0 + 2]], rows_a, sem_a)

        pltpu.make_async_copy(table.at[src_v.at[j1]], rows_b, sem_b).wait()
        pltpu.sync_copy(rows_b, acc.at[dst_v.at[j1]], add=True)
        return carry

    lax.fori_loop(0, K // 2, _pair, 0)

    plsc.subcore_barrier()
    pltpu.sync_copy(acc.at[rows], out.at[cid, rows])


def _mm_body(x_ref, w_ref, o_ref):
    o_ref[...] = jnp.dot(x_ref[...], w_ref[...],
                         preferred_element_type=jnp.float32)


def _fin_body(p_ref, w_ref, b_ref, o_ref):
    o_ref[...] = jnp.dot(p_ref[0] + p_ref[1], w_ref[...],
                         preferred_element_type=jnp.float32) + b_ref[...]


def kernel(adjacency, feature, W1, b1, W2, b2):
    edges = jnp.reshape(adjacency.astype(jnp.int32),
                        (2, REAL_CHUNKS, CHUNK))

    b1r = b1.reshape(1, D_HID).astype(jnp.float32)
    w2p = jnp.pad(W2.astype(jnp.float32),
                  ((0, 0), (0, D_HID - W2.shape[1])))
    b2p = jnp.pad(b2.astype(jnp.float32),
                  (0, D_HID - b2.shape[0])).reshape(1, D_HID)

    support1 = pl.pallas_call(
        _mm_body,
        out_shape=jax.ShapeDtypeStruct((N_NODES, D_HID), jnp.float32),
    )(feature, W1)

    part1 = _spmm(support1, edges)

    part2, _ = _spmm_relu(part1, edges, b1r)

    logits16 = pl.pallas_call(
        _fin_body,
        out_shape=jax.ShapeDtypeStruct((ACC_ROWS, D_HID), jnp.float32),
    )(part2, w2p, b2p)

    return logits16[:N_NODES, :7]


# speedup vs baseline: 1.0246x; 1.0246x over previous
import functools

import jax
import jax.numpy as jnp
from jax import lax
from jax.experimental import pallas as pl
from jax.experimental.pallas import tpu as pltpu
from jax.experimental.pallas import tpu_sc as plsc

N_NODES = 10000
N_EDGES = 320000
D_HID = 16

NUM_CORES = 2
NUM_SUBCORES = 16
NUM_WORKERS = NUM_CORES * NUM_SUBCORES
CHUNK = 1250
REAL_CHUNKS = N_EDGES // CHUNK
K = REAL_CHUNKS // NUM_WORKERS
ACC_ROWS = 10112
Z_ROWS = ACC_ROWS // NUM_SUBCORES

_MESH = plsc.VectorSubcoreMesh(core_axis_name="c", subcore_axis_name="s")


@functools.partial(
    pl.kernel,
    out_type=jax.ShapeDtypeStruct((NUM_CORES, ACC_ROWS, D_HID), jnp.float32),
    mesh=_MESH,
    scratch_types=[
        pltpu.VMEM_SHARED((ACC_ROWS, D_HID), jnp.float32),
        pltpu.VMEM((K, CHUNK), jnp.int32),
        pltpu.VMEM((K, CHUNK), jnp.int32),
        pltpu.VMEM((CHUNK, D_HID), jnp.float32),
        pltpu.VMEM((CHUNK, D_HID), jnp.float32),
        pltpu.VMEM((Z_ROWS, D_HID), jnp.float32),
        pltpu.SemaphoreType.DMA,
        pltpu.SemaphoreType.DMA,
    ],
    compiler_params=pltpu.CompilerParams(use_tc_tiling_on_sc=False),
)
def _spmm(table, edges, out, acc, src_v, dst_v, rows_a, rows_b, zero_v,
          sem_a, sem_b):
    cid = lax.axis_index("c")
    sid = lax.axis_index("s")
    wid = cid * NUM_SUBCORES + sid

    base = wid * K

    pltpu.async_copy(edges.at[0, pl.ds(base, K)], src_v, sem_a)
    pltpu.async_copy(edges.at[1, pl.ds(base, K)], dst_v, sem_b)

    def _zero(i, carry):
        zero_v[i, :] = jnp.zeros((D_HID,), jnp.float32)
        return carry

    lax.fori_loop(0, Z_ROWS, _zero, 0, unroll=8)
    pltpu.sync_copy(zero_v, acc.at[pl.ds(sid * Z_ROWS, Z_ROWS)])

    pltpu.make_async_copy(edges.at[0, pl.ds(base, K)], src_v, sem_a).wait()
    pltpu.make_async_copy(edges.at[1, pl.ds(base, K)], dst_v, sem_b).wait()
    plsc.subcore_barrier()

    pltpu.async_copy(table.at[src_v.at[0]], rows_a, sem_a)

    def _pair(i, carry):
        j0 = 2 * i
        j1 = j0 + 1
        pltpu.async_copy(table.at[src_v.at[j1]], rows_b, sem_b)
        pltpu.make_async_copy(table.at[src_v.at[j0]], rows_a, sem_a).wait()
        pltpu.sync_copy(rows_a, acc.at[dst_v.at[j0]], add=True)

        @pl.when(j0 + 2 < K)
        def _():
            pltpu.async_copy(table.at[src_v.at[j0 + 2]], rows_a, sem_a)

        pltpu.make_async_copy(table.at[src_v.at[j1]], rows_b, sem_b).wait()
        pltpu.sync_copy(rows_b, acc.at[dst_v.at[j1]], add=True)
        return carry

    lax.fori_loop(0, K // 2, _pair, 0)

    plsc.subcore_barrier()
    pltpu.sync_copy(
        acc.at[pl.ds(sid * Z_ROWS, Z_ROWS)],
        out.at[cid, pl.ds(sid * Z_ROWS, Z_ROWS)],
    )


@functools.partial(
    pl.kernel,
    out_type=[
        jax.ShapeDtypeStruct((NUM_CORES, ACC_ROWS, D_HID), jnp.float32),
        jax.ShapeDtypeStruct((NUM_CORES, ACC_ROWS, D_HID), jnp.float32),
    ],
    mesh=_MESH,
    scratch_types=[
        pltpu.VMEM_SHARED((ACC_ROWS, D_HID), jnp.float32),
        pltpu.VMEM((K, CHUNK), jnp.int32),
        pltpu.VMEM((K, CHUNK), jnp.int32),
        pltpu.VMEM((CHUNK, D_HID), jnp.float32),
        pltpu.VMEM((CHUNK, D_HID), jnp.float32),
        pltpu.VMEM((Z_ROWS, D_HID), jnp.float32),
        pltpu.VMEM((Z_ROWS, D_HID), jnp.float32),
        pltpu.VMEM((1, D_HID), jnp.float32),
        pltpu.SemaphoreType.DMA,
        pltpu.SemaphoreType.DMA,
    ],
    compiler_params=pltpu.CompilerParams(use_tc_tiling_on_sc=False),
)
def _spmm_relu(part1, edges, b1, out, htab, acc, src_v, dst_v, rows_a, rows_b,
               va, vb, b1_v, sem_a, sem_b):
    cid = lax.axis_index("c")
    sid = lax.axis_index("s")
    wid = cid * NUM_SUBCORES + sid

    base = wid * K
    pltpu.async_copy(edges.at[0, pl.ds(base, K)], src_v, sem_a)
    pltpu.async_copy(edges.at[1, pl.ds(base, K)], dst_v, sem_b)

    rows = pl.ds(sid * Z_ROWS, Z_ROWS)
    pltpu.sync_copy(b1.at[pl.ds(0, 1)], b1_v)
    pltpu.sync_copy(part1.at[0, rows], va)
    pltpu.sync_copy(part1.at[1, rows], vb)
    b1v = b1_v[0, :]

    def _mid(i, carry):
        vb[i, :] = jnp.maximum(va[i, :] + vb[i, :] + b1v, 0.0)
        va[i, :] = jnp.zeros((D_HID,), jnp.float32)
        return carry

    lax.fori_loop(0, Z_ROWS, _mid, 0, unroll=8)
    pltpu.sync_copy(vb, htab.at[cid, rows])
    pltpu.sync_copy(va, acc.at[rows])

    pltpu.make_async_copy(edges.at[0, pl.ds(base, K)], src_v, sem_a).wait()
    pltpu.make_async_copy(edges.at[1, pl.ds(base, K)], dst_v, sem_b).wait()
    plsc.subcore_barrier()

    table = htab.at[cid]
    pltpu.async_copy(table.at[src_v.at[0]], rows_a, sem_a)

    def _pair(i, carry):
        j0 = 2 * i
        j1 = j0 + 1
        pltpu.async_copy(table.at[src_v.at[j1]], rows_b, sem_b)
        pltpu.make_async_copy(table.at[src_v.at[j0]], rows_a, sem_a).wait()
        pltpu.sync_copy(rows_a, acc.at[dst_v.at[j0]], add=True)

        @pl.when(j0 + 2 < K)
        def _():
            pltpu.async_copy(table.at[src_v.at[j0 + 2]], rows_a, sem_a)

        pltpu.make_async_copy(table.at[src_v.at[j1]], rows_b, sem_b).wait()
        pltpu.sync_copy(rows_b, acc.at[dst_v.at[j1]], add=True)
        return carry

    lax.fori_loop(0, K // 2, _pair, 0)

    plsc.subcore_barrier()
    pltpu.sync_copy(acc.at[rows], out.at[cid, rows])


def _mm_body(x_ref, w_ref, o_ref):
    o_ref[...] = jnp.dot(x_ref[...], w_ref[...],
                         preferred_element_type=jnp.float32)


def _fin_body(p_ref, w_ref, b_ref, o_ref):
    o_ref[...] = jnp.dot(p_ref[0] + p_ref[1], w_ref[...],
                         preferred_element_type=jnp.float32) + b_ref[...]


def kernel(adjacency, feature, W1, b1, W2, b2):
    edges = jnp.reshape(adjacency.astype(jnp.int32),
                        (2, REAL_CHUNKS, CHUNK))

    b1r = b1.reshape(1, D_HID).astype(jnp.float32)
    w2p = jnp.pad(W2.astype(jnp.float32),
                  ((0, 0), (0, D_HID - W2.shape[1])))
    b2p = jnp.pad(b2.astype(jnp.float32),
                  (0, D_HID - b2.shape[0])).reshape(1, D_HID)

    support1 = pl.pallas_call(
        _mm_body,
        out_shape=jax.ShapeDtypeStruct((N_NODES, D_HID), jnp.float32),
    )(feature, W1)

    part1 = _spmm(support1, edges)

    part2, _ = _spmm_relu(part1, edges, b1r)

    logits16 = pl.pallas_call(
        _fin_body,
        out_shape=jax.ShapeDtypeStruct((ACC_ROWS, D_HID), jnp.float32),
    )(part2, w2p, b2p)

    return logits16[:N_NODES, :7]


# speedup vs baseline: 1.0676x; 1.0419x over previous
import functools

import jax
import jax.numpy as jnp
from jax import lax
from jax.experimental import pallas as pl
from jax.experimental.pallas import tpu as pltpu
from jax.experimental.pallas import tpu_sc as plsc

N_NODES = 10000
N_EDGES = 320000
D_HID = 16

NUM_CORES = 2
NUM_SUBCORES = 16
NUM_WORKERS = NUM_CORES * NUM_SUBCORES
CHUNK = 1000
REAL_CHUNKS = N_EDGES // CHUNK
K = REAL_CHUNKS // NUM_WORKERS
ACC_ROWS = 10112
Z_ROWS = ACC_ROWS // NUM_SUBCORES

_MESH = plsc.VectorSubcoreMesh(core_axis_name="c", subcore_axis_name="s")


@functools.partial(
    pl.kernel,
    out_type=jax.ShapeDtypeStruct((NUM_CORES, ACC_ROWS, D_HID), jnp.float32),
    mesh=_MESH,
    scratch_types=[
        pltpu.VMEM_SHARED((ACC_ROWS, D_HID), jnp.float32),
        pltpu.VMEM((K, CHUNK), jnp.int32),
        pltpu.VMEM((K, CHUNK), jnp.int32),
        pltpu.VMEM((CHUNK, D_HID), jnp.float32),
        pltpu.VMEM((CHUNK, D_HID), jnp.float32),
        pltpu.VMEM((Z_ROWS, D_HID), jnp.float32),
        pltpu.SemaphoreType.DMA,
        pltpu.SemaphoreType.DMA,
    ],
    compiler_params=pltpu.CompilerParams(use_tc_tiling_on_sc=False),
)
def _spmm(table, edges, out, acc, src_v, dst_v, rows_a, rows_b, zero_v,
          sem_a, sem_b):
    cid = lax.axis_index("c")
    sid = lax.axis_index("s")
    wid = cid * NUM_SUBCORES + sid

    base = wid * K

    pltpu.async_copy(edges.at[0, pl.ds(base, K)], src_v, sem_a)
    pltpu.async_copy(edges.at[1, pl.ds(base, K)], dst_v, sem_b)

    def _zero(i, carry):
        zero_v[i, :] = jnp.zeros((D_HID,), jnp.float32)
        return carry

    lax.fori_loop(0, Z_ROWS, _zero, 0, unroll=8)
    pltpu.sync_copy(zero_v, acc.at[pl.ds(sid * Z_ROWS, Z_ROWS)])

    pltpu.make_async_copy(edges.at[0, pl.ds(base, K)], src_v, sem_a).wait()
    pltpu.make_async_copy(edges.at[1, pl.ds(base, K)], dst_v, sem_b).wait()
    plsc.subcore_barrier()

    pltpu.async_copy(table.at[src_v.at[0]], rows_a, sem_a)

    def _pair(i, carry):
        j0 = 2 * i
        j1 = j0 + 1
        pltpu.async_copy(table.at[src_v.at[j1]], rows_b, sem_b)
        pltpu.make_async_copy(table.at[src_v.at[j0]], rows_a, sem_a).wait()
        pltpu.sync_copy(rows_a, acc.at[dst_v.at[j0]], add=True)

        @pl.when(j0 + 2 < K)
        def _():
            pltpu.async_copy(table.at[src_v.at[j0 + 2]], rows_a, sem_a)

        pltpu.make_async_copy(table.at[src_v.at[j1]], rows_b, sem_b).wait()
        pltpu.sync_copy(rows_b, acc.at[dst_v.at[j1]], add=True)
        return carry

    lax.fori_loop(0, K // 2, _pair, 0)

    plsc.subcore_barrier()
    pltpu.sync_copy(
        acc.at[pl.ds(sid * Z_ROWS, Z_ROWS)],
        out.at[cid, pl.ds(sid * Z_ROWS, Z_ROWS)],
    )


@functools.partial(
    pl.kernel,
    out_type=[
        jax.ShapeDtypeStruct((NUM_CORES, ACC_ROWS, D_HID), jnp.float32),
        jax.ShapeDtypeStruct((NUM_CORES, ACC_ROWS, D_HID), jnp.float32),
    ],
    mesh=_MESH,
    scratch_types=[
        pltpu.VMEM_SHARED((ACC_ROWS, D_HID), jnp.float32),
        pltpu.VMEM((K, CHUNK), jnp.int32),
        pltpu.VMEM((K, CHUNK), jnp.int32),
        pltpu.VMEM((CHUNK, D_HID), jnp.float32),
        pltpu.VMEM((CHUNK, D_HID), jnp.float32),
        pltpu.VMEM((Z_ROWS, D_HID), jnp.float32),
        pltpu.VMEM((Z_ROWS, D_HID), jnp.float32),
        pltpu.VMEM((1, D_HID), jnp.float32),
        pltpu.SemaphoreType.DMA,
        pltpu.SemaphoreType.DMA,
    ],
    compiler_params=pltpu.CompilerParams(use_tc_tiling_on_sc=False),
)
def _spmm_relu(part1, edges, b1, out, htab, acc, src_v, dst_v, rows_a, rows_b,
               va, vb, b1_v, sem_a, sem_b):
    cid = lax.axis_index("c")
    sid = lax.axis_index("s")
    wid = cid * NUM_SUBCORES + sid

    base = wid * K
    pltpu.async_copy(edges.at[0, pl.ds(base, K)], src_v, sem_a)
    pltpu.async_copy(edges.at[1, pl.ds(base, K)], dst_v, sem_b)

    rows = pl.ds(sid * Z_ROWS, Z_ROWS)
    pltpu.sync_copy(b1.at[pl.ds(0, 1)], b1_v)
    pltpu.sync_copy(part1.at[0, rows], va)
    pltpu.sync_copy(part1.at[1, rows], vb)
    b1v = b1_v[0, :]

    def _mid(i, carry):
        vb[i, :] = jnp.maximum(va[i, :] + vb[i, :] + b1v, 0.0)
        va[i, :] = jnp.zeros((D_HID,), jnp.float32)
        return carry

    lax.fori_loop(0, Z_ROWS, _mid, 0, unroll=8)
    pltpu.sync_copy(vb, htab.at[cid, rows])
    pltpu.sync_copy(va, acc.at[rows])

    pltpu.make_async_copy(edges.at[0, pl.ds(base, K)], src_v, sem_a).wait()
    pltpu.make_async_copy(edges.at[1, pl.ds(base, K)], dst_v, sem_b).wait()
    plsc.subcore_barrier()

    table = htab.at[cid]
    pltpu.async_copy(table.at[src_v.at[0]], rows_a, sem_a)

    def _pair(i, carry):
        j0 = 2 * i
        j1 = j0 + 1
        pltpu.async_copy(table.at[src_v.at[j1]], rows_b, sem_b)
        pltpu.make_async_copy(table.at[src_v.at[j0]], rows_a, sem_a).wait()
        pltpu.sync_copy(rows_a, acc.at[dst_v.at[j0]], add=True)

        @pl.when(j0 + 2 < K)
        def _():
            pltpu.async_copy(table.at[src_v.at[j0 + 2]], rows_a, sem_a)

        pltpu.make_async_copy(table.at[src_v.at[j1]], rows_b, sem_b).wait()
        pltpu.sync_copy(rows_b, acc.at[dst_v.at[j1]], add=True)
        return carry

    lax.fori_loop(0, K // 2, _pair, 0)

    plsc.subcore_barrier()
    pltpu.sync_copy(acc.at[rows], out.at[cid, rows])


def _mm_body(x_ref, w_ref, o_ref):
    o_ref[...] = jnp.dot(x_ref[...], w_ref[...],
                         preferred_element_type=jnp.float32)


def _fin_body(p_ref, w_ref, b_ref, o_ref):
    o_ref[...] = jnp.dot(p_ref[0] + p_ref[1], w_ref[...],
                         preferred_element_type=jnp.float32) + b_ref[...]


def kernel(adjacency, feature, W1, b1, W2, b2):
    edges = jnp.reshape(adjacency.astype(jnp.int32),
                        (2, REAL_CHUNKS, CHUNK))

    b1r = b1.reshape(1, D_HID).astype(jnp.float32)
    w2p = jnp.pad(W2.astype(jnp.float32),
                  ((0, 0), (0, D_HID - W2.shape[1])))
    b2p = jnp.pad(b2.astype(jnp.float32),
                  (0, D_HID - b2.shape[0])).reshape(1, D_HID)

    support1 = pl.pallas_call(
        _mm_body,
        out_shape=jax.ShapeDtypeStruct((N_NODES, D_HID), jnp.float32),
    )(feature, W1)

    part1 = _spmm(support1, edges)

    part2, _ = _spmm_relu(part1, edges, b1r)

    logits16 = pl.pallas_call(
        _fin_body,
        out_shape=jax.ShapeDtypeStruct((ACC_ROWS, D_HID), jnp.float32),
    )(part2, w2p, b2p)

    return logits16[:N_NODES, :7]


# speedup vs baseline: 1.0786x; 1.0104x over previous
import functools

import jax
import jax.numpy as jnp
from jax import lax
from jax.experimental import pallas as pl
from jax.experimental.pallas import tpu as pltpu
from jax.experimental.pallas import tpu_sc as plsc

N_NODES = 10000
N_EDGES = 320000
D_HID = 16

NUM_CORES = 2
NUM_SUBCORES = 16
NUM_WORKERS = NUM_CORES * NUM_SUBCORES
CHUNK = 1000
REAL_CHUNKS = N_EDGES // CHUNK
K = REAL_CHUNKS // NUM_WORKERS
ACC_ROWS = 10112
Z_ROWS = ACC_ROWS // NUM_SUBCORES

_MESH = plsc.VectorSubcoreMesh(core_axis_name="c", subcore_axis_name="s")


@functools.partial(
    pl.kernel,
    out_type=jax.ShapeDtypeStruct((NUM_CORES, ACC_ROWS, D_HID), jnp.float32),
    mesh=_MESH,
    scratch_types=[
        pltpu.VMEM_SHARED((ACC_ROWS, D_HID), jnp.float32),
        pltpu.VMEM((K, CHUNK), jnp.int32),
        pltpu.VMEM((K, CHUNK), jnp.int32),
        pltpu.VMEM((CHUNK, D_HID), jnp.float32),
        pltpu.VMEM((CHUNK, D_HID), jnp.float32),
        pltpu.VMEM((Z_ROWS, D_HID), jnp.float32),
        pltpu.SemaphoreType.DMA,
        pltpu.SemaphoreType.DMA,
    ],
    compiler_params=pltpu.CompilerParams(use_tc_tiling_on_sc=False),
)
def _spmm(table, edges, out, acc, src_v, dst_v, rows_a, rows_b, zero_v,
          sem_a, sem_b):
    cid = lax.axis_index("c")
    sid = lax.axis_index("s")
    wid = cid * NUM_SUBCORES + sid

    base = wid * K

    pltpu.async_copy(edges.at[0, pl.ds(base, K)], src_v, sem_a)
    pltpu.async_copy(edges.at[1, pl.ds(base, K)], dst_v, sem_b)

    def _zero(i, carry):
        zero_v[i, :] = jnp.zeros((D_HID,), jnp.float32)
        return carry

    lax.fori_loop(0, Z_ROWS, _zero, 0, unroll=8)
    pltpu.sync_copy(zero_v, acc.at[pl.ds(sid * Z_ROWS, Z_ROWS)])

    pltpu.make_async_copy(edges.at[0, pl.ds(base, K)], src_v, sem_a).wait()
    pltpu.make_async_copy(edges.at[1, pl.ds(base, K)], dst_v, sem_b).wait()
    plsc.subcore_barrier()

    pltpu.async_copy(table.at[src_v.at[0]], rows_a, sem_a)

    def _pair(i, carry):
        j0 = 2 * i
        j1 = j0 + 1
        pltpu.async_copy(table.at[src_v.at[j1]], rows_b, sem_b)
        pltpu.make_async_copy(table.at[src_v.at[j0]], rows_a, sem_a).wait()
        pltpu.sync_copy(rows_a, acc.at[dst_v.at[j0]], add=True)

        @pl.when(j0 + 2 < K)
        def _():
            pltpu.async_copy(table.at[src_v.at[j0 + 2]], rows_a, sem_a)

        pltpu.make_async_copy(table.at[src_v.at[j1]], rows_b, sem_b).wait()
        pltpu.sync_copy(rows_b, acc.at[dst_v.at[j1]], add=True)
        return carry

    lax.fori_loop(0, K // 2, _pair, 0)

    plsc.subcore_barrier()
    pltpu.sync_copy(
        acc.at[pl.ds(sid * Z_ROWS, Z_ROWS)],
        out.at[cid, pl.ds(sid * Z_ROWS, Z_ROWS)],
    )


@functools.partial(
    pl.kernel,
    out_type=[
        jax.ShapeDtypeStruct((NUM_CORES, ACC_ROWS, D_HID), jnp.float32),
        jax.ShapeDtypeStruct((NUM_CORES, ACC_ROWS, D_HID), jnp.float32),
    ],
    mesh=_MESH,
    scratch_types=[
        pltpu.VMEM_SHARED((ACC_ROWS, D_HID), jnp.float32),
        pltpu.VMEM((K, CHUNK), jnp.int32),
        pltpu.VMEM((K, CHUNK), jnp.int32),
        pltpu.VMEM((CHUNK, D_HID), jnp.float32),
        pltpu.VMEM((CHUNK, D_HID), jnp.float32),
        pltpu.VMEM((Z_ROWS, D_HID), jnp.float32),
        pltpu.VMEM((Z_ROWS, D_HID), jnp.float32),
        pltpu.VMEM((1, D_HID), jnp.float32),
        pltpu.SemaphoreType.DMA,
        pltpu.SemaphoreType.DMA,
    ],
    compiler_params=pltpu.CompilerParams(use_tc_tiling_on_sc=False),
)
def _spmm_relu(part1, edges, b1, out, htab, acc, src_v, dst_v, rows_a, rows_b,
               va, vb, b1_v, sem_a, sem_b):
    cid = lax.axis_index("c")
    sid = lax.axis_index("s")
    wid = cid * NUM_SUBCORES + sid

    base = wid * K
    pltpu.async_copy(edges.at[0, pl.ds(base, K)], src_v, sem_a)
    pltpu.async_copy(edges.at[1, pl.ds(base, K)], dst_v, sem_b)

    rows = pl.ds(sid * Z_ROWS, Z_ROWS)
    pltpu.sync_copy(b1.at[pl.ds(0, 1)], b1_v)
    pltpu.sync_copy(part1.at[0, rows], va)
    pltpu.sync_copy(part1.at[1, rows], vb)
    b1v = b1_v[0, :]

    def _mid(i, carry):
        vb[i, :] = jnp.maximum(va[i, :] + vb[i, :] + b1v, 0.0)
        va[i, :] = jnp.zeros((D_HID,), jnp.float32)
        return carry

    lax.fori_loop(0, Z_ROWS, _mid, 0, unroll=8)
    pltpu.sync_copy(vb, htab.at[cid, rows])
    pltpu.sync_copy(va, acc.at[rows])

    pltpu.make_async_copy(edges.at[0, pl.ds(base, K)], src_v, sem_a).wait()
    pltpu.make_async_copy(edges.at[1, pl.ds(base, K)], dst_v, sem_b).wait()
    plsc.subcore_barrier()

    table = htab.at[cid]
    pltpu.async_copy(table.at[src_v.at[0]], rows_a, sem_a)

    def _pair(i, carry):
        j0 = 2 * i
        j1 = j0 + 1
        pltpu.async_copy(table.at[src_v.at[j1]], rows_b, sem_b)
        pltpu.make_async_copy(table.at[src_v.at[j0]], rows_a, sem_a).wait()
        pltpu.sync_copy(rows_a, acc.at[dst_v.at[j0]], add=True)

        @pl.when(j0 + 2 < K)
        def _():
            pltpu.async_copy(table.at[src_v.at[j0 + 2]], rows_a, sem_a)

        pltpu.make_async_copy(table.at[src_v.at[j1]], rows_b, sem_b).wait()
        pltpu.sync_copy(rows_b, acc.at[dst_v.at[j1]], add=True)
        return carry

    lax.fori_loop(0, K // 2, _pair, 0)

    plsc.subcore_barrier()
    pltpu.sync_copy(acc.at[rows], out.at[cid, rows])


def _mm_body(x_ref, w_ref, o_ref):
    o_ref[...] = jnp.dot(x_ref[...], w_ref[...],
                         preferred_element_type=jnp.float32)


def _fin_body(p_ref, w_ref, b_ref, o_ref):
    r = jnp.dot(p_ref[0] + p_ref[1], w_ref[...],
                preferred_element_type=jnp.float32) + b_ref[...]
    o_ref[...] = r[:N_NODES, :7]


def kernel(adjacency, feature, W1, b1, W2, b2):
    edges = jnp.reshape(adjacency.astype(jnp.int32),
                        (2, REAL_CHUNKS, CHUNK))

    b1r = b1.reshape(1, D_HID).astype(jnp.float32)
    w2p = jnp.pad(W2.astype(jnp.float32),
                  ((0, 0), (0, D_HID - W2.shape[1])))
    b2p = jnp.pad(b2.astype(jnp.float32),
                  (0, D_HID - b2.shape[0])).reshape(1, D_HID)

    support1 = pl.pallas_call(
        _mm_body,
        out_shape=jax.ShapeDtypeStruct((N_NODES, D_HID), jnp.float32),
    )(feature, W1)

    part1 = _spmm(support1, edges)

    part2, _ = _spmm_relu(part1, edges, b1r)

    return pl.pallas_call(
        _fin_body,
        out_shape=jax.ShapeDtypeStruct((N_NODES, 7), jnp.float32),
    )(part2, w2p, b2p)


# speedup vs baseline: 1.0920x; 1.0124x over previous
import functools

import jax
import jax.numpy as jnp
from jax import lax
from jax.experimental import pallas as pl
from jax.experimental.pallas import tpu as pltpu
from jax.experimental.pallas import tpu_sc as plsc

N_NODES = 10000
N_EDGES = 320000
D_HID = 16

NUM_CORES = 2
NUM_SUBCORES = 16
NUM_WORKERS = NUM_CORES * NUM_SUBCORES
CHUNK = 1000
REAL_CHUNKS = N_EDGES // CHUNK
K = REAL_CHUNKS // NUM_WORKERS
ACC_ROWS = 10112
Z_ROWS = ACC_ROWS // NUM_SUBCORES

_MESH = plsc.VectorSubcoreMesh(core_axis_name="c", subcore_axis_name="s")


@functools.partial(
    pl.kernel,
    out_type=jax.ShapeDtypeStruct((NUM_CORES, ACC_ROWS, D_HID), jnp.float32),
    mesh=_MESH,
    scratch_types=[
        pltpu.VMEM_SHARED((ACC_ROWS, D_HID), jnp.float32),
        pltpu.VMEM((K, CHUNK), jnp.int32),
        pltpu.VMEM((K, CHUNK), jnp.int32),
        pltpu.VMEM((CHUNK, D_HID), jnp.float32),
        pltpu.VMEM((CHUNK, D_HID), jnp.float32),
        pltpu.VMEM((Z_ROWS, D_HID), jnp.float32),
        pltpu.SemaphoreType.DMA,
        pltpu.SemaphoreType.DMA,
    ],
    compiler_params=pltpu.CompilerParams(use_tc_tiling_on_sc=False),
)
def _spmm(table, edges, out, acc, src_v, dst_v, rows_a, rows_b, zero_v,
          sem_a, sem_b):
    cid = lax.axis_index("c")
    sid = lax.axis_index("s")
    wid = cid * NUM_SUBCORES + sid

    base = wid * K

    pltpu.async_copy(edges.at[0, pl.ds(base, K)], src_v, sem_a)
    pltpu.async_copy(edges.at[1, pl.ds(base, K)], dst_v, sem_b)

    pltpu.make_async_copy(edges.at[0, pl.ds(base, K)], src_v, sem_a).wait()
    pltpu.async_copy(table.at[src_v.at[0]], rows_a, sem_a)

    def _zero(i, carry):
        zero_v[i, :] = jnp.zeros((D_HID,), jnp.float32)
        return carry

    lax.fori_loop(0, Z_ROWS, _zero, 0, unroll=8)
    pltpu.sync_copy(zero_v, acc.at[pl.ds(sid * Z_ROWS, Z_ROWS)])

    pltpu.make_async_copy(edges.at[1, pl.ds(base, K)], dst_v, sem_b).wait()
    plsc.subcore_barrier()


    def _pair(i, carry):
        j0 = 2 * i
        j1 = j0 + 1
        pltpu.async_copy(table.at[src_v.at[j1]], rows_b, sem_b)
        pltpu.make_async_copy(table.at[src_v.at[j0]], rows_a, sem_a).wait()
        pltpu.sync_copy(rows_a, acc.at[dst_v.at[j0]], add=True)

        @pl.when(j0 + 2 < K)
        def _():
            pltpu.async_copy(table.at[src_v.at[j0 + 2]], rows_a, sem_a)

        pltpu.make_async_copy(table.at[src_v.at[j1]], rows_b, sem_b).wait()
        pltpu.sync_copy(rows_b, acc.at[dst_v.at[j1]], add=True)
        return carry

    lax.fori_loop(0, K // 2, _pair, 0)

    plsc.subcore_barrier()
    pltpu.sync_copy(
        acc.at[pl.ds(sid * Z_ROWS, Z_ROWS)],
        out.at[cid, pl.ds(sid * Z_ROWS, Z_ROWS)],
    )


@functools.partial(
    pl.kernel,
    out_type=[
        jax.ShapeDtypeStruct((NUM_CORES, ACC_ROWS, D_HID), jnp.float32),
        jax.ShapeDtypeStruct((NUM_CORES, ACC_ROWS, D_HID), jnp.float32),
    ],
    mesh=_MESH,
    scratch_types=[
        pltpu.VMEM_SHARED((ACC_ROWS, D_HID), jnp.float32),
        pltpu.VMEM((K, CHUNK), jnp.int32),
        pltpu.VMEM((K, CHUNK), jnp.int32),
        pltpu.VMEM((CHUNK, D_HID), jnp.float32),
        pltpu.VMEM((CHUNK, D_HID), jnp.float32),
        pltpu.VMEM((Z_ROWS, D_HID), jnp.float32),
        pltpu.VMEM((Z_ROWS, D_HID), jnp.float32),
        pltpu.VMEM((1, D_HID), jnp.float32),
        pltpu.SemaphoreType.DMA,
        pltpu.SemaphoreType.DMA,
    ],
    compiler_params=pltpu.CompilerParams(use_tc_tiling_on_sc=False),
)
def _spmm_relu(part1, edges, b1, out, htab, acc, src_v, dst_v, rows_a, rows_b,
               va, vb, b1_v, sem_a, sem_b):
    cid = lax.axis_index("c")
    sid = lax.axis_index("s")
    wid = cid * NUM_SUBCORES + sid

    base = wid * K
    pltpu.async_copy(edges.at[0, pl.ds(base, K)], src_v, sem_a)
    pltpu.async_copy(edges.at[1, pl.ds(base, K)], dst_v, sem_b)

    rows = pl.ds(sid * Z_ROWS, Z_ROWS)
    pltpu.sync_copy(b1.at[pl.ds(0, 1)], b1_v)
    pltpu.sync_copy(part1.at[0, rows], va)
    pltpu.sync_copy(part1.at[1, rows], vb)
    b1v = b1_v[0, :]

    def _mid(i, carry):
        vb[i, :] = jnp.maximum(va[i, :] + vb[i, :] + b1v, 0.0)
        va[i, :] = jnp.zeros((D_HID,), jnp.float32)
        return carry

    lax.fori_loop(0, Z_ROWS, _mid, 0, unroll=8)
    pltpu.sync_copy(vb, htab.at[cid, rows])
    pltpu.sync_copy(va, acc.at[rows])

    pltpu.make_async_copy(edges.at[0, pl.ds(base, K)], src_v, sem_a).wait()
    pltpu.make_async_copy(edges.at[1, pl.ds(base, K)], dst_v, sem_b).wait()
    plsc.subcore_barrier()

    table = htab.at[cid]
    pltpu.async_copy(table.at[src_v.at[0]], rows_a, sem_a)

    def _pair(i, carry):
        j0 = 2 * i
        j1 = j0 + 1
        pltpu.async_copy(table.at[src_v.at[j1]], rows_b, sem_b)
        pltpu.make_async_copy(table.at[src_v.at[j0]], rows_a, sem_a).wait()
        pltpu.sync_copy(rows_a, acc.at[dst_v.at[j0]], add=True)

        @pl.when(j0 + 2 < K)
        def _():
            pltpu.async_copy(table.at[src_v.at[j0 + 2]], rows_a, sem_a)

        pltpu.make_async_copy(table.at[src_v.at[j1]], rows_b, sem_b).wait()
        pltpu.sync_copy(rows_b, acc.at[dst_v.at[j1]], add=True)
        return carry

    lax.fori_loop(0, K // 2, _pair, 0)

    plsc.subcore_barrier()
    pltpu.sync_copy(acc.at[rows], out.at[cid, rows])


def _mm_body(x_ref, w_ref, o_ref):
    o_ref[...] = jnp.dot(x_ref[...], w_ref[...],
                         preferred_element_type=jnp.float32)


def _fin_body(p_ref, w_ref, b_ref, o_ref):
    r = jnp.dot(p_ref[0] + p_ref[1], w_ref[...],
                preferred_element_type=jnp.float32) + b_ref[...]
    o_ref[...] = r[:N_NODES, :7]


def kernel(adjacency, feature, W1, b1, W2, b2):
    edges = jnp.reshape(adjacency.astype(jnp.int32),
                        (2, REAL_CHUNKS, CHUNK))

    b1r = b1.reshape(1, D_HID).astype(jnp.float32)
    w2p = jnp.pad(W2.astype(jnp.float32),
                  ((0, 0), (0, D_HID - W2.shape[1])))
    b2p = jnp.pad(b2.astype(jnp.float32),
                  (0, D_HID - b2.shape[0])).reshape(1, D_HID)

    support1 = pl.pallas_call(
        _mm_body,
        out_shape=jax.ShapeDtypeStruct((N_NODES, D_HID), jnp.float32),
    )(feature, W1)

    part1 = _spmm(support1, edges)

    part2, _ = _spmm_relu(part1, edges, b1r)

    return pl.pallas_call(
        _fin_body,
        out_shape=jax.ShapeDtypeStruct((N_NODES, 7), jnp.float32),
    )(part2, w2p, b2p)
